# Initial kernel scaffold; baseline (speedup 1.0000x reference)
import jax
import jax.numpy as jnp
from jax.experimental import pallas as pl


def kernel(x, W1, b1, W2, b2, Wg1, bg1, Wg2, bg2, temperature):
    raise NotImplementedError("write your pallas kernel here")


# speedup vs baseline: 1.9025x; 1.9025x over previous
import jax
import jax.numpy as jnp
from jax.experimental import pallas as pl
from jax.experimental.pallas import tpu as pltpu


def _gates_kernel(t_ref, x_ref, Wg1_ref, bg1_ref, Wg2_ref, bg2_ref, gates_ref):
    x = x_ref[...].astype(jnp.bfloat16)
    gh = jnp.dot(x, Wg1_ref[...].astype(jnp.bfloat16),
                 preferred_element_type=jnp.float32)
    gh = jnp.maximum(gh + bg1_ref[...], 0.0)
    logits = jnp.dot(gh.astype(jnp.bfloat16),
                     Wg2_ref[...].astype(jnp.bfloat16),
                     preferred_element_type=jnp.float32)
    logits = (logits + bg2_ref[...]) / t_ref[0]

    e_dim = logits.shape[-1]
    iota = jax.lax.broadcasted_iota(jnp.int32, logits.shape, 1)
    m1 = jnp.max(logits, axis=-1, keepdims=True)
    eq1 = logits == m1
    idx1 = jnp.min(jnp.where(eq1, iota, e_dim), axis=-1, keepdims=True)
    first = iota == idx1
    l2 = jnp.where(first, -jnp.inf, logits)
    m2 = jnp.max(l2, axis=-1, keepdims=True)
    eq2 = l2 == m2
    idx2 = jnp.min(jnp.where(eq2, iota, e_dim), axis=-1, keepdims=True)
    second = iota == idx2
    b = jnp.exp(m2 - m1)
    denom = 1.0 + b
    g1 = 1.0 / denom
    g2 = b / denom
    gates_ref[...] = jnp.where(first, g1, 0.0) + jnp.where(second, g2, 0.0)


def _experts_kernel(x_ref, W1_ref, b1_ref, W2_ref, b2_ref, gates_ref, out_ref):
    e = pl.program_id(0)
    x = x_ref[...]
    h = jnp.dot(x, W1_ref[0], precision=jax.lax.Precision.DEFAULT,
                preferred_element_type=jnp.float32)
    h = jnp.maximum(h + b1_ref[0], 0.0)
    y = jnp.dot(h, W2_ref[0], precision=jax.lax.Precision.DEFAULT,
                preferred_element_type=jnp.float32)
    y = y + b2_ref[0]
    eiota = jax.lax.broadcasted_iota(jnp.int32, gates_ref.shape, 1)
    g = jnp.sum(jnp.where(eiota == e, gates_ref[...], 0.0), axis=1,
                keepdims=True)
    contrib = g * y

    @pl.when(e == 0)
    def _():
        out_ref[...] = contrib

    @pl.when(e > 0)
    def _():
        out_ref[...] += contrib


def kernel(x, W1, b1, W2, b2, Wg1, bg1, Wg2, bg2, temperature):
    n, d = x.shape
    e_num, _, h_dim = W1.shape
    t = jnp.reshape(temperature.astype(jnp.float32), (1,))
    bg1_2d = jnp.reshape(bg1, (1, h_dim))
    bg2_2d = jnp.reshape(bg2, (1, e_num))
    b1_3d = jnp.reshape(b1, (e_num, 1, h_dim))
    b2_3d = jnp.reshape(b2, (e_num, 1, h_dim))

    nb_gate = 512
    gates = pl.pallas_call(
        _gates_kernel,
        grid=(n // nb_gate,),
        in_specs=[
            pl.BlockSpec(memory_space=pltpu.SMEM),
            pl.BlockSpec((nb_gate, d), lambda i: (i, 0)),
            pl.BlockSpec((d, h_dim), lambda i: (0, 0)),
            pl.BlockSpec((1, h_dim), lambda i: (0, 0)),
            pl.BlockSpec((h_dim, e_num), lambda i: (0, 0)),
            pl.BlockSpec((1, e_num), lambda i: (0, 0)),
        ],
        out_specs=pl.BlockSpec((nb_gate, e_num), lambda i: (i, 0)),
        out_shape=jax.ShapeDtypeStruct((n, e_num), jnp.float32),
        compiler_params=pltpu.CompilerParams(
            dimension_semantics=("arbitrary",),
        ),
    )(t, x, Wg1, bg1_2d, Wg2, bg2_2d)

    out = pl.pallas_call(
        _experts_kernel,
        grid=(e_num,),
        in_specs=[
            pl.BlockSpec((n, d), lambda e: (0, 0)),
            pl.BlockSpec((1, d, h_dim), lambda e: (e, 0, 0)),
            pl.BlockSpec((1, 1, h_dim), lambda e: (e, 0, 0)),
            pl.BlockSpec((1, h_dim, h_dim), lambda e: (e, 0, 0)),
            pl.BlockSpec((1, 1, h_dim), lambda e: (e, 0, 0)),
            pl.BlockSpec((n, e_num), lambda e: (0, 0)),
        ],
        out_specs=pl.BlockSpec((n, h_dim), lambda e: (0, 0)),
        out_shape=jax.ShapeDtypeStruct((n, h_dim), jnp.float32),
        compiler_params=pltpu.CompilerParams(
            dimension_semantics=("arbitrary",),
        ),
    )(x, W1, b1_3d, W2, b2_3d, gates)

    return out, gates
